# baseline (speedup 1.0000x reference)
import jax
import jax.numpy as jnp
from jax import lax
from jax.experimental import pallas as pl
from jax.experimental.pallas import tpu as pltpu
from jax.experimental.pallas import tpu_sc as plsc

NUM_EMB = 1_000_000
DIM = 64
BATCH = 16384

_INFO = plsc.get_sparse_core_info()
_NC = _INFO.num_cores
_NS = _INFO.num_subcores
_NW = _NC * _NS

_FULLB = NUM_EMB // 128
_TAIL0 = _FULLB * 128
_BASEB = _FULLB // _NW
_EXTRA = _FULLB % _NW
_CPB = 4
_NCHUNK = -(-(_BASEB + 1) // _CPB)
_CAP = 2048
_NGRP = BATCH // 16


def _body(idx_hbm, tab_hbm, tail_hbm, out_hbm,
          idxall, posm, rm, pc, rc, chunk3, extflat, sem_blk, sem_out):
    wid = lax.axis_index("s") * _NC + lax.axis_index("c")
    lob = wid * _BASEB + jnp.minimum(wid, _EXTRA)
    nb = jnp.where(wid < _EXTRA, _BASEB + 1, _BASEB)
    lo = lob * 128
    hi = jnp.where(wid == _NW - 1, jnp.int32(1 << 30), (lob + nb) * 128)
    iota = lax.iota(jnp.int32, 16)
    nchunks = _NCHUNK + jnp.where(wid == _NW - 1, 1, 0)

    def issue_chunk(c):
        buf = c % 2
        is_tail = c == _NCHUNK

        @pl.when(is_tail)
        def _():
            pltpu.async_copy(tail_hbm, chunk3.at[buf, 0], sem_blk)
            for bi in range(1, _CPB):
                pltpu.async_copy(
                    tab_hbm.at[:, pl.ds(0, 128)], chunk3.at[buf, bi], sem_blk)

        @pl.when(jnp.logical_not(is_tail))
        def _():
            for bi in range(_CPB):
                j = jnp.minimum(lob + c * _CPB + bi, lob + nb - 1)
                pltpu.async_copy(
                    tab_hbm.at[:, pl.ds(pl.multiple_of(j * 128, 128), 128)],
                    chunk3.at[buf, bi], sem_blk)

    issue_chunk(jnp.int32(0))

    pltpu.sync_copy(idx_hbm, idxall)

    def scan(g, cnt):
        v = idxall[pl.ds(g * 16, 16)]
        m = (v >= lo) & (v < hi)
        mi = m.astype(jnp.int32)
        csum = plsc.cumsum(mi)
        ranks = csum - mi
        plsc.store_scatter(posm, [cnt + ranks], iota + g * 16, mask=m)
        plsc.store_scatter(rm, [cnt + ranks], v, mask=m)
        return cnt + csum[15]

    cnt = lax.fori_loop(0, _NGRP, scan, jnp.int32(0))
    npages = (cnt + (_CAP - 1)) // _CAP

    def chunk_body(c, gg):
        buf = c % 2
        is_tail = c == _NCHUNK
        clo = jnp.where(is_tail, jnp.int32(_TAIL0), (lob + c * _CPB) * 128)
        chi = jnp.where(is_tail, jnp.int32(1 << 30),
                        jnp.minimum(clo + _CPB * 128, (lob + nb) * 128))

        for bi in range(_CPB):
            pltpu.make_async_copy(
                tab_hbm.at[:, pl.ds(0, 128)], chunk3.at[buf, bi], sem_blk
            ).wait()

        @pl.when(c + 1 < nchunks)
        def _():
            issue_chunk(c + 1)

        bufv = jnp.zeros((16,), jnp.int32) + buf

        def page_body(p, gg):
            def cscan(gi, cntc):
                b0 = p * _CAP + gi * 16
                rv = rm[pl.ds(b0, 16)]
                pv = posm[pl.ds(b0, 16)]
                mv = (b0 + iota < cnt) & (rv >= clo) & (rv < chi)
                mvi = mv.astype(jnp.int32)
                csum = plsc.cumsum(mvi)
                rks = csum - mvi
                plsc.store_scatter(rc, [cntc + rks], rv, mask=mv)
                plsc.store_scatter(pc, [cntc + rks], pv, mask=mv)
                return cntc + csum[15]

            pcnt = jnp.minimum(cnt - p * _CAP, _CAP)
            cntc = lax.fori_loop(0, (pcnt + 15) // 16, cscan, jnp.int32(0))
            rc[pl.ds(cntc, 16)] = jnp.zeros((16,), jnp.int32) + clo
            pc[pl.ds(cntc, 16)] = jnp.zeros((16,), jnp.int32) + BATCH
            egroups = (cntc + 15) // 16

            def egroup(e, gg):
                pc16 = pc[pl.ds(e * 16, 16)]
                rc16 = rc[pl.ds(e * 16, 16)]
                rloc = rc16 - clo
                bv = rloc >> 7
                rlv = rloc & 127
                slotbase = (gg % 8) * 16
                for l in range(16):
                    bb = jnp.zeros((16,), jnp.int32) + bv[l]
                    rr = jnp.zeros((16,), jnp.int32) + rlv[l]
                    s64 = (slotbase + l) * 64
                    for cb in range(4):
                        vals = plsc.load_gather(
                            chunk3, [bufv, bb, iota + cb * 16, rr])
                        extflat[pl.ds(s64 + cb * 16, 16)] = vals
                    pltpu.async_copy(
                        extflat.at[pl.ds(s64, 64)],
                        out_hbm.at[pl.ds(pc16[l] * 64, 64)], sem_out)
                gg = gg + 1

                @pl.when(gg > 7)
                def _():
                    pltpu.make_async_copy(
                        out_hbm.at[pl.ds(0, 1024)],
                        extflat.at[pl.ds(0, 1024)], sem_out).wait()

                return gg

            return lax.fori_loop(0, egroups, egroup, gg)

        return lax.fori_loop(0, npages, page_body, gg)

    gg = lax.fori_loop(0, nchunks, chunk_body, jnp.int32(0))

    def final_drain(d, carry):
        pltpu.make_async_copy(
            out_hbm.at[pl.ds(0, 1024)],
            extflat.at[pl.ds(0, 1024)], sem_out).wait()
        return carry

    lax.fori_loop(0, jnp.minimum(gg, 7), final_drain, jnp.int32(0))


@jax.jit
def kernel(inputs, weights):
    k = pl.kernel(
        _body,
        out_type=jax.ShapeDtypeStruct((BATCH * DIM + DIM,), jnp.float32),
        mesh=plsc.VectorSubcoreMesh(core_axis_name="c", subcore_axis_name="s"),
        scratch_types=[
            pltpu.VMEM((BATCH,), jnp.int32),
            pltpu.VMEM((BATCH,), jnp.int32),
            pltpu.VMEM((BATCH,), jnp.int32),
            pltpu.VMEM((_CAP + 16,), jnp.int32),
            pltpu.VMEM((_CAP + 16,), jnp.int32),
            pltpu.VMEM((2, _CPB, DIM, 128), jnp.float32),
            pltpu.VMEM((128 * DIM,), jnp.float32),
            pltpu.SemaphoreType.DMA,
            pltpu.SemaphoreType.DMA,
        ],
        compiler_params=pltpu.CompilerParams(needs_layout_passes=False),
    )
    tail_t = jnp.pad(weights[_TAIL0:].T, ((0, 0), (0, 128 - (NUM_EMB - _TAIL0))))
    flat = k(inputs, weights.T, tail_t)
    return flat[: BATCH * DIM].reshape(BATCH, DIM)


# speedup vs baseline: 1.5985x; 1.5985x over previous
import jax
import jax.numpy as jnp
from jax import lax
from jax.experimental import pallas as pl
from jax.experimental.pallas import tpu as pltpu
from jax.experimental.pallas import tpu_sc as plsc

NUM_EMB = 1_000_000
DIM = 64
BATCH = 16384

_INFO = plsc.get_sparse_core_info()
_NC = _INFO.num_cores
_NS = _INFO.num_subcores
_NW = _NC * _NS
_BPW = BATCH // _NW


def _gather_body(idx_hbm, table_hbm, out_hbm, idx_v, rows_v, sem):
    wid = lax.axis_index("s") * _NC + lax.axis_index("c")
    base = wid * _BPW
    pltpu.sync_copy(idx_hbm.at[pl.ds(base, _BPW)], idx_v)

    def issue(g, carry):
        vec = idx_v[pl.ds(g * 16, 16)]
        for l in range(16):
            r = vec[l]
            pltpu.async_copy(
                table_hbm.at[pl.ds(r, 1)], rows_v.at[pl.ds(g * 16 + l, 1)], sem
            )
        return carry

    lax.fori_loop(0, _BPW // 16, issue, 0)

    def drain(j, carry):
        pltpu.make_async_copy(
            table_hbm.at[pl.ds(0, 1)], rows_v.at[pl.ds(j, 1)], sem
        ).wait()
        return carry

    lax.fori_loop(0, _BPW, drain, 0)
    pltpu.sync_copy(rows_v, out_hbm.at[pl.ds(base, _BPW)])


@jax.jit
def kernel(inputs, weights):
    k = pl.kernel(
        _gather_body,
        out_type=jax.ShapeDtypeStruct((BATCH, DIM), jnp.float32),
        mesh=plsc.VectorSubcoreMesh(core_axis_name="c", subcore_axis_name="s"),
        scratch_types=[
            pltpu.VMEM((_BPW,), jnp.int32),
            pltpu.VMEM((_BPW, DIM), jnp.float32),
            pltpu.SemaphoreType.DMA,
        ],
    )
    return k(inputs, weights)
